# Initial kernel scaffold; baseline (speedup 1.0000x reference)
import jax
import jax.numpy as jnp
from jax.experimental import pallas as pl


def kernel(x, Wg, bg, W1, b1, W2, b2, Wh, bh):
    raise NotImplementedError("write your pallas kernel here")


# speedup vs baseline: 1.0519x; 1.0519x over previous
import functools

import jax
import jax.numpy as jnp
from jax.experimental import pallas as pl
from jax.experimental.pallas import tpu as pltpu

MAX_DEPTH = 2
NUM_EXPERTS = 16
D_MODEL = 1024
HIDDEN = 1024
THRESHOLD = 0.8


def _moe_act_kernel(
    x_ref, wg_ref, bg_ref, w1_ref, b1_ref, w2_ref, b2_ref, wh_ref, bh_ref,
    out_ref, nupd_ref, rem_ref, depth_ref,
    cur_ref, acc_ref, gate_ref, hp_ref, active_ref,
):
    s = pl.program_id(0)
    e = pl.program_id(1)

    @pl.when(jnp.logical_and(s == 0, e == 0))
    def _init():
        cur_ref[...] = x_ref[...]
        out_ref[...] = jnp.zeros_like(out_ref)
        nupd_ref[...] = jnp.zeros_like(nupd_ref)
        rem_ref[...] = jnp.zeros_like(rem_ref)
        depth_ref[...] = jnp.zeros_like(depth_ref)
        hp_ref[...] = jnp.zeros_like(hp_ref)
        active_ref[...] = jnp.ones_like(active_ref)

    @pl.when(e == 0)
    def _gate():
        logits = (
            jnp.dot(cur_ref[...], wg_ref[0], preferred_element_type=jnp.float32)
            + bg_ref[0, 0]
        )
        m = jnp.max(logits, axis=-1, keepdims=True)
        ex = jnp.exp(logits - m)
        gate_ref[...] = ex / jnp.sum(ex, axis=-1, keepdims=True)
        acc_ref[...] = jnp.zeros_like(acc_ref)

    h = jnp.maximum(
        jnp.dot(cur_ref[...], w1_ref[0, 0], preferred_element_type=jnp.float32)
        + b1_ref[0, 0, 0],
        0.0,
    )
    eo = (
        jnp.dot(h, w2_ref[0, 0], preferred_element_type=jnp.float32)
        + b2_ref[0, 0, 0]
    )
    lane = jax.lax.broadcasted_iota(jnp.int32, (1, NUM_EXPERTS), 1)
    g_col = jnp.sum(
        jnp.where(lane == e, gate_ref[...], 0.0), axis=-1, keepdims=True
    )
    acc_ref[...] += g_col * eo

    @pl.when(e == NUM_EXPERTS - 1)
    def _halt():
        cur = cur_ref[...]
        wh_col = wh_ref[0, :, 0]
        p = jax.nn.sigmoid(
            jnp.sum(cur * wh_col[None, :], axis=-1, keepdims=True) + bh_ref[0, 0, 0]
        )
        sr = active_ref[...]
        hp = hp_ref[...]
        new_halted = jnp.where(hp + p * sr >= THRESHOLD, 1.0, 0.0) * sr
        sr2 = sr - new_halted
        inc = new_halted * (THRESHOLD - hp)
        hp_new = hp + p * sr2 + inc
        uw = p * sr2 + inc
        rem_ref[...] += new_halted * (1.0 - hp_new)
        out_ref[...] = out_ref[...] * (1.0 - uw) + acc_ref[...] * uw
        depth_ref[...] += sr
        nupd_ref[...] += uw
        hp_ref[...] = hp_new
        active_ref[...] = jnp.where(hp_new < THRESHOLD, 1.0, 0.0)
        cur_ref[...] = out_ref[...]


@jax.jit
def kernel(x, Wg, bg, W1, b1, W2, b2, Wh, bh):
    B = x.shape[0]
    bg = bg.reshape(MAX_DEPTH, 1, NUM_EXPERTS)
    b1 = b1.reshape(MAX_DEPTH, NUM_EXPERTS, 1, HIDDEN)
    b2 = b2.reshape(MAX_DEPTH, NUM_EXPERTS, 1, D_MODEL)
    bh = bh.reshape(MAX_DEPTH, 1, 1)
    grid = (MAX_DEPTH, NUM_EXPERTS)
    out, nupd, rem, depth = pl.pallas_call(
        _moe_act_kernel,
        grid=grid,
        in_specs=[
            pl.BlockSpec((B, D_MODEL), lambda s, e: (0, 0)),
            pl.BlockSpec((1, D_MODEL, NUM_EXPERTS), lambda s, e: (s, 0, 0)),
            pl.BlockSpec((1, 1, NUM_EXPERTS), lambda s, e: (s, 0, 0)),
            pl.BlockSpec((1, 1, D_MODEL, HIDDEN), lambda s, e: (s, e, 0, 0)),
            pl.BlockSpec((1, 1, 1, HIDDEN), lambda s, e: (s, e, 0, 0)),
            pl.BlockSpec((1, 1, HIDDEN, D_MODEL), lambda s, e: (s, e, 0, 0)),
            pl.BlockSpec((1, 1, 1, D_MODEL), lambda s, e: (s, e, 0, 0)),
            pl.BlockSpec((1, D_MODEL, 1), lambda s, e: (s, 0, 0)),
            pl.BlockSpec((1, 1, 1), lambda s, e: (s, 0, 0)),
        ],
        out_specs=[
            pl.BlockSpec((B, D_MODEL), lambda s, e: (0, 0)),
            pl.BlockSpec((B, 1), lambda s, e: (0, 0)),
            pl.BlockSpec((B, 1), lambda s, e: (0, 0)),
            pl.BlockSpec((B, 1), lambda s, e: (0, 0)),
        ],
        out_shape=[
            jax.ShapeDtypeStruct((B, D_MODEL), jnp.float32),
            jax.ShapeDtypeStruct((B, 1), jnp.float32),
            jax.ShapeDtypeStruct((B, 1), jnp.float32),
            jax.ShapeDtypeStruct((B, 1), jnp.float32),
        ],
        scratch_shapes=[
            pltpu.VMEM((B, D_MODEL), jnp.float32),
            pltpu.VMEM((B, D_MODEL), jnp.float32),
            pltpu.VMEM((B, NUM_EXPERTS), jnp.float32),
            pltpu.VMEM((B, 1), jnp.float32),
            pltpu.VMEM((B, 1), jnp.float32),
        ],
        compiler_params=pltpu.CompilerParams(
            dimension_semantics=("arbitrary", "arbitrary"),
        ),
    )(x, Wg, bg, W1, b1, W2, b2, Wh, bh)
    return (out, nupd[:, 0], rem[:, 0], depth[:, 0])


# speedup vs baseline: 1.0628x; 1.0103x over previous
import functools

import jax
import jax.numpy as jnp
from jax.experimental import pallas as pl
from jax.experimental.pallas import tpu as pltpu

MAX_DEPTH = 2
NUM_EXPERTS = 16
D_MODEL = 1024
HIDDEN = 1024
THRESHOLD = 0.8


def _moe_act_kernel(
    x_ref, wg_ref, bg_ref, w1a_ref, w1b_ref, b1_ref, w2a_ref, w2b_ref,
    b2_ref, wh_ref, bh_ref,
    out_ref, nupd_ref, rem_ref, depth_ref,
    cur_ref, acc_ref, gate_ref, hp_ref, active_ref,
):
    s = pl.program_id(0)
    e = pl.program_id(1)

    @pl.when(jnp.logical_and(s == 0, e == 0))
    def _init():
        cur_ref[...] = x_ref[...]
        out_ref[...] = jnp.zeros_like(out_ref)
        nupd_ref[...] = jnp.zeros_like(nupd_ref)
        rem_ref[...] = jnp.zeros_like(rem_ref)
        depth_ref[...] = jnp.zeros_like(depth_ref)
        hp_ref[...] = jnp.zeros_like(hp_ref)
        active_ref[...] = jnp.ones_like(active_ref)

    @pl.when(e == 0)
    def _gate():
        logits = (
            jnp.dot(cur_ref[...], wg_ref[0], preferred_element_type=jnp.float32)
            + bg_ref[0, 0]
        )
        m = jnp.max(logits, axis=-1, keepdims=True)
        ex = jnp.exp(logits - m)
        gate_ref[...] = ex / jnp.sum(ex, axis=-1, keepdims=True)
        acc_ref[...] = jnp.zeros_like(acc_ref)

    cur = cur_ref[...]
    b1_row = b1_ref[0, 0, 0]
    ha = jnp.maximum(
        jnp.dot(cur, w1a_ref[0, 0], preferred_element_type=jnp.float32)
        + b1_row[: HIDDEN // 2],
        0.0,
    )
    hb = jnp.maximum(
        jnp.dot(cur, w1b_ref[0, 0], preferred_element_type=jnp.float32)
        + b1_row[HIDDEN // 2 :],
        0.0,
    )
    eo = (
        jnp.dot(ha, w2a_ref[0, 0], preferred_element_type=jnp.float32)
        + jnp.dot(hb, w2b_ref[0, 0], preferred_element_type=jnp.float32)
        + b2_ref[0, 0, 0]
    )
    lane = jax.lax.broadcasted_iota(jnp.int32, (1, NUM_EXPERTS), 1)
    g_col = jnp.sum(
        jnp.where(lane == e, gate_ref[...], 0.0), axis=-1, keepdims=True
    )
    acc_ref[...] += g_col * eo

    @pl.when(e == NUM_EXPERTS - 1)
    def _halt():
        cur = cur_ref[...]
        wh_col = wh_ref[0, :, 0]
        p = jax.nn.sigmoid(
            jnp.sum(cur * wh_col[None, :], axis=-1, keepdims=True) + bh_ref[0, 0, 0]
        )
        sr = active_ref[...]
        hp = hp_ref[...]
        new_halted = jnp.where(hp + p * sr >= THRESHOLD, 1.0, 0.0) * sr
        sr2 = sr - new_halted
        inc = new_halted * (THRESHOLD - hp)
        hp_new = hp + p * sr2 + inc
        uw = p * sr2 + inc
        rem_ref[...] += new_halted * (1.0 - hp_new)
        out_ref[...] = out_ref[...] * (1.0 - uw) + acc_ref[...] * uw
        depth_ref[...] += sr
        nupd_ref[...] += uw
        hp_ref[...] = hp_new
        active_ref[...] = jnp.where(hp_new < THRESHOLD, 1.0, 0.0)
        cur_ref[...] = out_ref[...]


@jax.jit
def kernel(x, Wg, bg, W1, b1, W2, b2, Wh, bh):
    B = x.shape[0]
    bg = bg.reshape(MAX_DEPTH, 1, NUM_EXPERTS)
    b1 = b1.reshape(MAX_DEPTH, NUM_EXPERTS, 1, HIDDEN)
    b2 = b2.reshape(MAX_DEPTH, NUM_EXPERTS, 1, D_MODEL)
    bh = bh.reshape(MAX_DEPTH, 1, 1)
    grid = (MAX_DEPTH, NUM_EXPERTS)
    out, nupd, rem, depth = pl.pallas_call(
        _moe_act_kernel,
        grid=grid,
        in_specs=[
            pl.BlockSpec((B, D_MODEL), lambda s, e: (0, 0)),
            pl.BlockSpec((1, D_MODEL, NUM_EXPERTS), lambda s, e: (s, 0, 0)),
            pl.BlockSpec((1, 1, NUM_EXPERTS), lambda s, e: (s, 0, 0)),
            pl.BlockSpec((1, 1, D_MODEL, HIDDEN // 2), lambda s, e: (s, e, 0, 0)),
            pl.BlockSpec((1, 1, D_MODEL, HIDDEN // 2), lambda s, e: (s, e, 0, 1)),
            pl.BlockSpec((1, 1, 1, HIDDEN), lambda s, e: (s, e, 0, 0)),
            pl.BlockSpec((1, 1, HIDDEN // 2, D_MODEL), lambda s, e: (s, e, 0, 0)),
            pl.BlockSpec((1, 1, HIDDEN // 2, D_MODEL), lambda s, e: (s, e, 1, 0)),
            pl.BlockSpec((1, 1, 1, D_MODEL), lambda s, e: (s, e, 0, 0)),
            pl.BlockSpec((1, D_MODEL, 1), lambda s, e: (s, 0, 0)),
            pl.BlockSpec((1, 1, 1), lambda s, e: (s, 0, 0)),
        ],
        out_specs=[
            pl.BlockSpec((B, D_MODEL), lambda s, e: (0, 0)),
            pl.BlockSpec((B, 1), lambda s, e: (0, 0)),
            pl.BlockSpec((B, 1), lambda s, e: (0, 0)),
            pl.BlockSpec((B, 1), lambda s, e: (0, 0)),
        ],
        out_shape=[
            jax.ShapeDtypeStruct((B, D_MODEL), jnp.float32),
            jax.ShapeDtypeStruct((B, 1), jnp.float32),
            jax.ShapeDtypeStruct((B, 1), jnp.float32),
            jax.ShapeDtypeStruct((B, 1), jnp.float32),
        ],
        scratch_shapes=[
            pltpu.VMEM((B, D_MODEL), jnp.float32),
            pltpu.VMEM((B, D_MODEL), jnp.float32),
            pltpu.VMEM((B, NUM_EXPERTS), jnp.float32),
            pltpu.VMEM((B, 1), jnp.float32),
            pltpu.VMEM((B, 1), jnp.float32),
        ],
        compiler_params=pltpu.CompilerParams(
            dimension_semantics=("arbitrary", "arbitrary"),
        ),
    )(x, Wg, bg, W1, W1, b1, W2, W2, b2, Wh, bh)
    return (out, nupd[:, 0], rem[:, 0], depth[:, 0])


# speedup vs baseline: 1.0769x; 1.0132x over previous
import functools

import jax
import jax.numpy as jnp
from jax.experimental import pallas as pl
from jax.experimental.pallas import tpu as pltpu

MAX_DEPTH = 2
NUM_EXPERTS = 16
D_MODEL = 1024
HIDDEN = 1024
THRESHOLD = 0.8


NSPLIT = 4


def _moe_act_kernel(
    x_ref, wg_ref, bg_ref, *rest,
):
    w1_refs = rest[:NSPLIT]
    b1_ref = rest[NSPLIT]
    w2_refs = rest[NSPLIT + 1 : 2 * NSPLIT + 1]
    b2_ref, wh_ref, bh_ref = rest[2 * NSPLIT + 1 : 2 * NSPLIT + 4]
    out_ref, nupd_ref, rem_ref, depth_ref = rest[2 * NSPLIT + 4 : 2 * NSPLIT + 8]
    cur_ref, acc_ref, gate_ref, hp_ref, active_ref = rest[2 * NSPLIT + 8 :]
    s = pl.program_id(0)
    e = pl.program_id(1)

    @pl.when(jnp.logical_and(s == 0, e == 0))
    def _init():
        cur_ref[...] = x_ref[...]
        out_ref[...] = jnp.zeros_like(out_ref)
        nupd_ref[...] = jnp.zeros_like(nupd_ref)
        rem_ref[...] = jnp.zeros_like(rem_ref)
        depth_ref[...] = jnp.zeros_like(depth_ref)
        hp_ref[...] = jnp.zeros_like(hp_ref)
        active_ref[...] = jnp.ones_like(active_ref)

    @pl.when(e == 0)
    def _gate():
        logits = (
            jnp.dot(cur_ref[...], wg_ref[0], preferred_element_type=jnp.float32)
            + bg_ref[0, 0]
        )
        m = jnp.max(logits, axis=-1, keepdims=True)
        ex = jnp.exp(logits - m)
        gate_ref[...] = ex / jnp.sum(ex, axis=-1, keepdims=True)
        acc_ref[...] = jnp.zeros_like(acc_ref)

    cur = cur_ref[...]
    b1_row = b1_ref[0, 0, 0]
    hs = HIDDEN // NSPLIT
    eo = b2_ref[0, 0, 0]
    for k in range(NSPLIT):
        hk = jnp.maximum(
            jnp.dot(cur, w1_refs[k][0, 0], preferred_element_type=jnp.float32)
            + b1_row[k * hs : (k + 1) * hs],
            0.0,
        )
        eo = eo + jnp.dot(
            hk, w2_refs[k][0, 0], preferred_element_type=jnp.float32
        )
    lane = jax.lax.broadcasted_iota(jnp.int32, (1, NUM_EXPERTS), 1)
    g_col = jnp.sum(
        jnp.where(lane == e, gate_ref[...], 0.0), axis=-1, keepdims=True
    )
    acc_ref[...] += g_col * eo

    @pl.when(e == NUM_EXPERTS - 1)
    def _halt():
        cur = cur_ref[...]
        wh_col = wh_ref[0, :, 0]
        p = jax.nn.sigmoid(
            jnp.sum(cur * wh_col[None, :], axis=-1, keepdims=True) + bh_ref[0, 0, 0]
        )
        sr = active_ref[...]
        hp = hp_ref[...]
        new_halted = jnp.where(hp + p * sr >= THRESHOLD, 1.0, 0.0) * sr
        sr2 = sr - new_halted
        inc = new_halted * (THRESHOLD - hp)
        hp_new = hp + p * sr2 + inc
        uw = p * sr2 + inc
        rem_ref[...] += new_halted * (1.0 - hp_new)
        out_ref[...] = out_ref[...] * (1.0 - uw) + acc_ref[...] * uw
        depth_ref[...] += sr
        nupd_ref[...] += uw
        hp_ref[...] = hp_new
        active_ref[...] = jnp.where(hp_new < THRESHOLD, 1.0, 0.0)
        cur_ref[...] = out_ref[...]


@jax.jit
def kernel(x, Wg, bg, W1, b1, W2, b2, Wh, bh):
    B = x.shape[0]
    bg = bg.reshape(MAX_DEPTH, 1, NUM_EXPERTS)
    b1 = b1.reshape(MAX_DEPTH, NUM_EXPERTS, 1, HIDDEN)
    b2 = b2.reshape(MAX_DEPTH, NUM_EXPERTS, 1, D_MODEL)
    bh = bh.reshape(MAX_DEPTH, 1, 1)
    grid = (MAX_DEPTH, NUM_EXPERTS)
    out, nupd, rem, depth = pl.pallas_call(
        _moe_act_kernel,
        grid=grid,
        in_specs=[
            pl.BlockSpec((B, D_MODEL), lambda s, e: (0, 0)),
            pl.BlockSpec((1, D_MODEL, NUM_EXPERTS), lambda s, e: (s, 0, 0)),
            pl.BlockSpec((1, 1, NUM_EXPERTS), lambda s, e: (s, 0, 0)),
            *[
                pl.BlockSpec(
                    (1, 1, D_MODEL, HIDDEN // NSPLIT),
                    functools.partial(lambda k, s, e: (s, e, 0, k), k),
                )
                for k in range(NSPLIT)
            ],
            pl.BlockSpec((1, 1, 1, HIDDEN), lambda s, e: (s, e, 0, 0)),
            *[
                pl.BlockSpec(
                    (1, 1, HIDDEN // NSPLIT, D_MODEL),
                    functools.partial(lambda k, s, e: (s, e, k, 0), k),
                )
                for k in range(NSPLIT)
            ],
            pl.BlockSpec((1, 1, 1, D_MODEL), lambda s, e: (s, e, 0, 0)),
            pl.BlockSpec((1, D_MODEL, 1), lambda s, e: (s, 0, 0)),
            pl.BlockSpec((1, 1, 1), lambda s, e: (s, 0, 0)),
        ],
        out_specs=[
            pl.BlockSpec((B, D_MODEL), lambda s, e: (0, 0)),
            pl.BlockSpec((B, 1), lambda s, e: (0, 0)),
            pl.BlockSpec((B, 1), lambda s, e: (0, 0)),
            pl.BlockSpec((B, 1), lambda s, e: (0, 0)),
        ],
        out_shape=[
            jax.ShapeDtypeStruct((B, D_MODEL), jnp.float32),
            jax.ShapeDtypeStruct((B, 1), jnp.float32),
            jax.ShapeDtypeStruct((B, 1), jnp.float32),
            jax.ShapeDtypeStruct((B, 1), jnp.float32),
        ],
        scratch_shapes=[
            pltpu.VMEM((B, D_MODEL), jnp.float32),
            pltpu.VMEM((B, D_MODEL), jnp.float32),
            pltpu.VMEM((B, NUM_EXPERTS), jnp.float32),
            pltpu.VMEM((B, 1), jnp.float32),
            pltpu.VMEM((B, 1), jnp.float32),
        ],
        compiler_params=pltpu.CompilerParams(
            dimension_semantics=("arbitrary", "arbitrary"),
        ),
    )(x, Wg, bg, *([W1] * NSPLIT), b1, *([W2] * NSPLIT), b2, Wh, bh)
    return (out, nupd[:, 0], rem[:, 0], depth[:, 0])
